# baseline (speedup 1.0000x reference)
import jax
import jax.numpy as jnp
from jax import lax
from jax.experimental import pallas as pl
from jax.experimental.pallas import tpu as pltpu
from jax.experimental.pallas import tpu_sc as plsc

N = 10000
E = 160000
K = 3
H = 256
IN = 8

NC = 2
NS = 16
CHUNK = 104
NSLOT = 3
TPT = E // NS
GRP = 6
NCHUNK = 102
TPT_PAD = NCHUNK * CHUNK
NGRP = NCHUNK // GRP
DCH = 128
EPT = E // (NC * NS)
DCHUNKS = -(-EPT // DCH)
NPAD = 10240
ROWS_PT = NPAD // NS
HH = H // 2
DRN = 64
NDRAIN = ROWS_PT // DRN

_mesh = plsc.VectorSubcoreMesh(core_axis_name="c", subcore_axis_name="s")


def _sc_deg_body(col_hbm, zeros_hbm, ones_hbm, deg_out, colv, ones_v, accum):
    cid = lax.axis_index("c")
    sid = lax.axis_index("s")
    base = sid * ROWS_PT
    pltpu.sync_copy(zeros_hbm.at[pl.ds(base, ROWS_PT)],
                    accum.at[pl.ds(base, ROWS_PT)])
    pltpu.sync_copy(col_hbm.at[cid, sid], colv)
    pltpu.sync_copy(ones_hbm, ones_v)
    plsc.subcore_barrier()

    @pl.loop(0, DCHUNKS)
    def _chunk(j):
        pltpu.sync_copy(ones_v, accum.at[colv.at[j]], add=True)

    plsc.subcore_barrier()
    pltpu.sync_copy(accum.at[pl.ds(base, ROWS_PT)],
                    deg_out.at[cid, pl.ds(base, ROWS_PT)])


_sc_deg = pl.kernel(
    _sc_deg_body,
    out_type=jax.ShapeDtypeStruct((NC, NPAD, HH), jnp.float32),
    mesh=_mesh,
    scratch_types=[
        pltpu.VMEM((DCHUNKS, DCH), jnp.int32),
        pltpu.VMEM((DCH, HH), jnp.float32),
        pltpu.VMEM_SHARED((NPAD, HH), jnp.float32),
    ],
)


def _make_hops(l0):
    def tile_body(cid, sid, args):
        (row_hbm, col_hbm, uin_hbm, d2_hbm, uout,
         irow, icol, gbuf0, gbuf1, gbuf2, d2v, accum,
         gsem0, gsem1, gsem2, ssem0, ssem1, ssem2, isem) = args
        base = sid * ROWS_PT
        gbufs = (gbuf0, gbuf1, gbuf2)
        gsems = (gsem0, gsem1, gsem2)
        ssems = (ssem0, ssem1, ssem2)

        def memset_zero(buf):
            @pl.loop(0, CHUNK)
            def _z(r):
                for v in range(HH // 16):
                    buf[r, pl.ds(v * 16, 16)] = jnp.zeros((16,), jnp.float32)

        memset_zero(gbuf2)

        @pl.loop(0, NDRAIN)
        def _z0(d):
            pltpu.sync_copy(gbuf2.at[pl.ds(0, DRN)],
                            accum.at[pl.ds(base + d * DRN, DRN)])

        plsc.subcore_barrier()
        for j in range(K):
            if j == 0:
                src = uin_hbm if l0 else uin_hbm.at[cid]
            else:
                src = uout.at[j - 1] if l0 else uout.at[cid, j - 1]

            pltpu.sync_copy(row_hbm.at[sid, 0], irow.at[0])
            pltpu.sync_copy(col_hbm.at[sid, 0], icol.at[0])

            @pl.loop(0, NGRP)
            def _grp(g):
                p = lax.rem(g, 3)
                pn = lax.rem(g + 1, 3)
                gn = jnp.minimum(g + 1, NGRP - 1)
                id0 = pltpu.async_copy(row_hbm.at[sid, gn],
                                       irow.at[pn], isem)
                id1 = pltpu.async_copy(col_hbm.at[sid, gn],
                                       icol.at[pn], isem)
                gd = [None] * GRP
                sd = [None] * GRP
                for b in range(NSLOT):
                    @pl.when(g > 0)
                    def _w():
                        pltpu.make_async_copy(
                            gbufs[b], accum.at[icol.at[p, b]],
                            ssems[b]).wait()
                    gd[b] = pltpu.async_copy(src.at[irow.at[p, b]],
                                             gbufs[b], gsems[b])
                for b in range(NSLOT):
                    gd[b].wait()
                    sd[b] = pltpu.async_copy(gbufs[b],
                                             accum.at[icol.at[p, b]],
                                             ssems[b], add=True)
                for b in range(NSLOT, GRP):
                    sd[b - NSLOT].wait()
                    gd[b] = pltpu.async_copy(src.at[irow.at[p, b]],
                                             gbufs[b % NSLOT],
                                             gsems[b % NSLOT])
                for b in range(NSLOT, GRP):
                    gd[b].wait()
                    sd[b] = pltpu.async_copy(gbufs[b % NSLOT],
                                             accum.at[icol.at[p, b]],
                                             ssems[b % NSLOT], add=True)
                id0.wait()
                id1.wait()

            for s in range(NSLOT):
                pltpu.make_async_copy(gbufs[s], accum.at[icol.at[0, s]],
                                      ssems[s]).wait()
            plsc.subcore_barrier()
            memset_zero(gbuf2)

            @pl.loop(0, NDRAIN)
            def _drain(d):
                db = base + d * DRN
                pltpu.sync_copy(accum.at[pl.ds(db, DRN)],
                                gbuf0.at[pl.ds(0, DRN)])
                pltpu.sync_copy(d2_hbm.at[pl.ds(db * 16, DRN * 16)], d2v)

                @pl.loop(0, DRN, unroll=2)
                def _scale(r):
                    dv = d2v[pl.ds(r * 16, 16)]
                    for v in range(1 if l0 else HH // 16):
                        gbuf0[r, pl.ds(v * 16, 16)] = (
                            gbuf0[r, pl.ds(v * 16, 16)] * dv)

                if j < K - 1:
                    pltpu.sync_copy(gbuf2.at[pl.ds(0, DRN)],
                                    accum.at[pl.ds(db, DRN)])
                dst = (uout.at[j, pl.ds(db, DRN)] if l0
                       else uout.at[cid, j, pl.ds(db, DRN)])
                pltpu.sync_copy(gbuf0.at[pl.ds(0, DRN)], dst)

            plsc.subcore_barrier()

    def body(*args):
        cid = lax.axis_index("c")
        sid = lax.axis_index("s")
        if l0:
            @pl.when(cid == 0)
            def _():
                tile_body(cid, sid, args)
        else:
            tile_body(cid, sid, args)

    scratch = [
        pltpu.VMEM((3, GRP, CHUNK), jnp.int32),
        pltpu.VMEM((3, GRP, CHUNK), jnp.int32),
        pltpu.VMEM((CHUNK, HH), jnp.float32),
        pltpu.VMEM((CHUNK, HH), jnp.float32),
        pltpu.VMEM((CHUNK, HH), jnp.float32),
        pltpu.VMEM((DRN * 16,), jnp.float32),
        pltpu.VMEM_SHARED((NPAD, HH), jnp.float32),
    ] + [pltpu.SemaphoreType.DMA] * 7
    oshape = (K, NPAD, HH) if l0 else (NC, K, NPAD, HH)
    return pl.kernel(
        body,
        out_type=jax.ShapeDtypeStruct(oshape, jnp.float32),
        mesh=_mesh,
        scratch_types=scratch,
    )


_sc_hops_l0 = _make_hops(l0=True)
_sc_hops = _make_hops(l0=False)


_PBLK = NPAD // 8
_TGRID = 8


def _tc_prep_body(deg2_ref, xp_ref, d2w_ref, sdeg_ref, dinv_ref, u0_ref):
    deg = deg2_ref[0, :, 0:1] + deg2_ref[1, :, 0:1]
    pos = deg > 0
    safe = jnp.maximum(deg, 1e-12)
    dinv = jnp.where(pos, lax.rsqrt(safe), 0.0)
    d2 = jnp.where(pos, 1.0 / safe, 0.0)
    d2w_ref[...] = jnp.broadcast_to(d2, (_PBLK, 16))
    sdeg_ref[...] = jnp.sqrt(deg)
    dinv_ref[...] = dinv
    u0_ref[...] = xp_ref[...] * dinv


_tc_prep = pl.pallas_call(
    _tc_prep_body,
    grid=(_TGRID,),
    in_specs=[
        pl.BlockSpec((NC, _PBLK, HH), lambda i: (0, i, 0)),
        pl.BlockSpec((_PBLK, HH), lambda i: (i, 0)),
    ],
    out_specs=[
        pl.BlockSpec((_PBLK, 16), lambda i: (i, 0)),
        pl.BlockSpec((_PBLK, 1), lambda i: (i, 0)),
        pl.BlockSpec((_PBLK, 1), lambda i: (i, 0)),
        pl.BlockSpec((_PBLK, HH), lambda i: (i, 0)),
    ],
    out_shape=[
        jax.ShapeDtypeStruct((NPAD, 16), jnp.float32),
        jax.ShapeDtypeStruct((NPAD, 1), jnp.float32),
        jax.ShapeDtypeStruct((NPAD, 1), jnp.float32),
        jax.ShapeDtypeStruct((NPAD, HH), jnp.float32),
    ],
)


def _tc_l0_body(xp_ref, u0_ref, sdeg_ref, dinv_ref, w_ref, b_ref,
                h_ref, u_ref):
    xs = xp_ref[:, :IN]
    acc = jnp.dot(xs, w_ref[0], preferred_element_type=jnp.float32)
    sd = sdeg_ref[...]
    for k in range(1, K + 1):
        uk = u0_ref[k - 1][:, :IN] * sd
        acc = acc + jnp.dot(uk, w_ref[k], preferred_element_type=jnp.float32)
    h = jnp.maximum(acc + b_ref[...], 0.0)
    h_ref[...] = h
    dv = dinv_ref[...]
    u_ref[0] = h[:, :HH] * dv
    u_ref[1] = h[:, HH:] * dv


_tc_l0 = pl.pallas_call(
    _tc_l0_body,
    grid=(_TGRID,),
    in_specs=[
        pl.BlockSpec((_PBLK, HH), lambda i: (i, 0)),
        pl.BlockSpec((K, _PBLK, HH), lambda i: (0, i, 0)),
        pl.BlockSpec((_PBLK, 1), lambda i: (i, 0)),
        pl.BlockSpec((_PBLK, 1), lambda i: (i, 0)),
        pl.BlockSpec((K + 1, IN, H), lambda i: (0, 0, 0)),
        pl.BlockSpec((1, H), lambda i: (0, 0)),
    ],
    out_specs=[
        pl.BlockSpec((_PBLK, H), lambda i: (i, 0)),
        pl.BlockSpec((NC, _PBLK, HH), lambda i: (0, i, 0)),
    ],
    out_shape=[
        jax.ShapeDtypeStruct((NPAD, H), jnp.float32),
        jax.ShapeDtypeStruct((NC, NPAD, HH), jnp.float32),
    ],
)


def _tc_mid_body(h_ref, u_ref, sdeg_ref, dinv_ref, w_ref, b_ref,
                 h_out, u_out):
    acc = jnp.dot(h_ref[...], w_ref[0], preferred_element_type=jnp.float32)
    sd = sdeg_ref[...]
    for k in range(1, K + 1):
        ua = u_ref[0, k - 1] * sd
        ub = u_ref[1, k - 1] * sd
        acc = acc + jnp.dot(ua, w_ref[k, :HH, :],
                            preferred_element_type=jnp.float32)
        acc = acc + jnp.dot(ub, w_ref[k, HH:, :],
                            preferred_element_type=jnp.float32)
    h = jnp.maximum(acc + b_ref[...], 0.0)
    h_out[...] = h
    dv = dinv_ref[...]
    u_out[0] = h[:, :HH] * dv
    u_out[1] = h[:, HH:] * dv


def _tc_fin_body(h_ref, u_ref, sdeg_ref, wf_ref, bf_ref, w_ref, b_ref,
                 y_ref):
    acc = jnp.dot(h_ref[...], w_ref[0], preferred_element_type=jnp.float32)
    sd = sdeg_ref[...]
    for k in range(1, K + 1):
        ua = u_ref[0, k - 1] * sd
        ub = u_ref[1, k - 1] * sd
        acc = acc + jnp.dot(ua, w_ref[k, :HH, :],
                            preferred_element_type=jnp.float32)
        acc = acc + jnp.dot(ub, w_ref[k, HH:, :],
                            preferred_element_type=jnp.float32)
    h = jnp.maximum(acc + b_ref[...], 0.0)
    y = jnp.dot(h, wf_ref[...], preferred_element_type=jnp.float32)
    y_ref[...] = jax.nn.sigmoid(y + bf_ref[...])


_mid_specs = [
    pl.BlockSpec((_PBLK, H), lambda i: (i, 0)),
    pl.BlockSpec((NC, K, _PBLK, HH), lambda i: (0, 0, i, 0)),
    pl.BlockSpec((_PBLK, 1), lambda i: (i, 0)),
]
_w_specs = [
    pl.BlockSpec((K + 1, H, H), lambda i: (0, 0, 0)),
    pl.BlockSpec((1, H), lambda i: (0, 0)),
]

_tc_mid = pl.pallas_call(
    _tc_mid_body,
    grid=(_TGRID,),
    in_specs=_mid_specs + [pl.BlockSpec((_PBLK, 1), lambda i: (i, 0))]
    + _w_specs,
    out_specs=[
        pl.BlockSpec((_PBLK, H), lambda i: (i, 0)),
        pl.BlockSpec((NC, _PBLK, HH), lambda i: (0, i, 0)),
    ],
    out_shape=[
        jax.ShapeDtypeStruct((NPAD, H), jnp.float32),
        jax.ShapeDtypeStruct((NC, NPAD, HH), jnp.float32),
    ],
)

_tc_fin = pl.pallas_call(
    _tc_fin_body,
    grid=(_TGRID,),
    in_specs=_mid_specs + [
        pl.BlockSpec((H, 1), lambda i: (0, 0)),
        pl.BlockSpec((1, 1), lambda i: (0, 0)),
    ] + _w_specs,
    out_specs=[pl.BlockSpec((_PBLK, 1), lambda i: (i, 0))],
    out_shape=[jax.ShapeDtypeStruct((NPAD, 1), jnp.float32)],
)


def kernel(x, edge_index, edge_attr, batch, W0, b0, W1, b1, W2, b2, Wf, bf):
    row = edge_index[0].reshape(NS, TPT)
    col = edge_index[1].reshape(NS, TPT)
    padi = jnp.zeros((NS, TPT_PAD - TPT), jnp.int32)
    rowp = jnp.concatenate([row, padi], axis=1).reshape(
        NS, NGRP, GRP, CHUNK)
    colp = jnp.concatenate([col, padi + N], axis=1).reshape(
        NS, NGRP, GRP, CHUNK)
    cold = edge_index[1].reshape(NC, NS, EPT)
    padd = jnp.full((NC, NS, DCHUNKS * DCH - EPT), N, jnp.int32)
    colpd = jnp.concatenate([cold, padd], axis=2).reshape(
        NC, NS, DCHUNKS, DCH)
    xp = jnp.zeros((NPAD, HH), x.dtype).at[:N, :IN].set(x)
    zeros128 = jnp.zeros((NPAD, HH), jnp.float32)
    ones128 = jnp.ones((DCH, HH), jnp.float32)

    deg2 = _sc_deg(colpd, zeros128, ones128)
    d2w, sdeg, dinv, u0p = _tc_prep(deg2, xp)
    d2flat = d2w.reshape(-1)

    u0k = _sc_hops_l0(rowp, colp, u0p, d2flat)
    h1, u1 = _tc_l0(xp, u0k, sdeg, dinv, W0, b0.reshape(1, H))

    u1k = _sc_hops(rowp, colp, u1, d2flat)
    h2, u2 = _tc_mid(h1, u1k, sdeg, dinv, W1, b1.reshape(1, H))

    u2k = _sc_hops(rowp, colp, u2, d2flat)
    (y,) = _tc_fin(h2, u2k, sdeg, Wf, bf.reshape(1, 1), W2, b2.reshape(1, H))

    return y[:N]


# speedup vs baseline: 2.5498x; 2.5498x over previous
import jax
import jax.numpy as jnp
from jax import lax
from jax.experimental import pallas as pl
from jax.experimental.pallas import tpu as pltpu
from jax.experimental.pallas import tpu_sc as plsc

N = 10000
E = 160000
K = 3
H = 256
IN = 8

NC = 2
NS = 16
CHUNK = 112
TPT = E // NS
GRP = 6
NCHUNK = 90
TPT_PAD = NCHUNK * CHUNK
NGRP = NCHUNK // GRP
DCH = 128
EPT = E // (NC * NS)
DCHUNKS = -(-EPT // DCH)
NPAD = 10240
ROWS_PT = NPAD // NS
HH = H // 2
DRN = 80
NDRAIN = ROWS_PT // DRN

_mesh = plsc.VectorSubcoreMesh(core_axis_name="c", subcore_axis_name="s")


def _sc_deg_body(col_hbm, zeros_hbm, ones_hbm, deg_out, colv, ones_v, accum):
    cid = lax.axis_index("c")
    sid = lax.axis_index("s")
    base = sid * ROWS_PT
    pltpu.sync_copy(zeros_hbm.at[pl.ds(base, ROWS_PT)],
                    accum.at[pl.ds(base, ROWS_PT)])
    pltpu.sync_copy(col_hbm.at[cid, sid], colv)
    pltpu.sync_copy(ones_hbm, ones_v)
    plsc.subcore_barrier()

    @pl.loop(0, DCHUNKS)
    def _chunk(j):
        pltpu.sync_copy(ones_v, accum.at[colv.at[j]], add=True)

    plsc.subcore_barrier()
    pltpu.sync_copy(accum.at[pl.ds(base, ROWS_PT)],
                    deg_out.at[cid, pl.ds(base, ROWS_PT)])


_sc_deg = pl.kernel(
    _sc_deg_body,
    out_type=jax.ShapeDtypeStruct((NC, NPAD, HH), jnp.float32),
    mesh=_mesh,
    scratch_types=[
        pltpu.VMEM((DCHUNKS, DCH), jnp.int32),
        pltpu.VMEM((DCH, HH), jnp.float32),
        pltpu.VMEM_SHARED((NPAD, HH), jnp.float32),
    ],
)


def _make_hops(l0):
    def tile_body(cid, sid, args):
        (row_hbm, col_hbm, uin_hbm, d2_hbm, uout,
         irow, icol, gbuf0, gbuf1, gbuf2, d2v, accum,
         gsem0, gsem1, gsem2, ssem0, ssem1, ssem2) = args
        base = sid * ROWS_PT
        gbufs = (gbuf0, gbuf1, gbuf2)
        gsems = (gsem0, gsem1, gsem2)
        ssems = (ssem0, ssem1, ssem2)

        def memset_zero(buf):
            @pl.loop(0, CHUNK)
            def _z(r):
                for v in range(HH // 16):
                    buf[r, pl.ds(v * 16, 16)] = jnp.zeros((16,), jnp.float32)

        memset_zero(gbuf2)

        @pl.loop(0, NDRAIN)
        def _z0(d):
            pltpu.sync_copy(gbuf2.at[pl.ds(0, DRN)],
                            accum.at[pl.ds(base + d * DRN, DRN)])

        plsc.subcore_barrier()
        for j in range(K):
            if j == 0:
                src = uin_hbm if l0 else uin_hbm.at[cid]
            else:
                src = uout.at[j - 1] if l0 else uout.at[cid, j - 1]

            @pl.loop(0, NGRP)
            def _grp(g):
                p = g & 1
                pltpu.sync_copy(row_hbm.at[sid, g], irow.at[p])
                pltpu.sync_copy(col_hbm.at[sid, g], icol.at[p])
                gd = [None] * GRP
                sd = [None] * GRP
                for b in range(3):
                    @pl.when(g > 0)
                    def _w():
                        pltpu.make_async_copy(
                            gbufs[b], accum.at[icol.at[p, b]],
                            ssems[b]).wait()
                    gd[b] = pltpu.async_copy(src.at[irow.at[p, b]],
                                             gbufs[b], gsems[b])
                for b in range(3):
                    gd[b].wait()
                    sd[b] = pltpu.async_copy(gbufs[b],
                                             accum.at[icol.at[p, b]],
                                             ssems[b], add=True)
                for b in range(3, GRP):
                    sd[b - 3].wait()
                    gd[b] = pltpu.async_copy(src.at[irow.at[p, b]],
                                             gbufs[b % 3], gsems[b % 3])
                for b in range(3, GRP):
                    gd[b].wait()
                    sd[b] = pltpu.async_copy(gbufs[b % 3],
                                             accum.at[icol.at[p, b]],
                                             ssems[b % 3], add=True)

            for s in range(3):
                pltpu.make_async_copy(gbufs[s], accum.at[icol.at[0, s]],
                                      ssems[s]).wait()
            plsc.subcore_barrier()
            memset_zero(gbuf2)

            @pl.loop(0, NDRAIN)
            def _drain(d):
                db = base + d * DRN
                pltpu.sync_copy(accum.at[pl.ds(db, DRN)],
                                gbuf0.at[pl.ds(0, DRN)])
                pltpu.sync_copy(d2_hbm.at[pl.ds(db * 16, DRN * 16)], d2v)

                @pl.loop(0, DRN, unroll=2)
                def _scale(r):
                    dv = d2v[pl.ds(r * 16, 16)]
                    for v in range(1 if l0 else HH // 16):
                        gbuf0[r, pl.ds(v * 16, 16)] = (
                            gbuf0[r, pl.ds(v * 16, 16)] * dv)

                if j < K - 1:
                    pltpu.sync_copy(gbuf2.at[pl.ds(0, DRN)],
                                    accum.at[pl.ds(db, DRN)])
                dst = (uout.at[j, pl.ds(db, DRN)] if l0
                       else uout.at[cid, j, pl.ds(db, DRN)])
                pltpu.sync_copy(gbuf0.at[pl.ds(0, DRN)], dst)

            plsc.subcore_barrier()

    def body(*args):
        cid = lax.axis_index("c")
        sid = lax.axis_index("s")
        if l0:
            @pl.when(cid == 0)
            def _():
                tile_body(cid, sid, args)
        else:
            tile_body(cid, sid, args)

    scratch = [
        pltpu.VMEM((2, GRP, CHUNK), jnp.int32),
        pltpu.VMEM((2, GRP, CHUNK), jnp.int32),
        pltpu.VMEM((CHUNK, HH), jnp.float32),
        pltpu.VMEM((CHUNK, HH), jnp.float32),
        pltpu.VMEM((CHUNK, HH), jnp.float32),
        pltpu.VMEM((DRN * 16,), jnp.float32),
        pltpu.VMEM_SHARED((NPAD, HH), jnp.float32),
        pltpu.SemaphoreType.DMA,
        pltpu.SemaphoreType.DMA,
        pltpu.SemaphoreType.DMA,
        pltpu.SemaphoreType.DMA,
        pltpu.SemaphoreType.DMA,
        pltpu.SemaphoreType.DMA,
    ]
    oshape = (K, NPAD, HH) if l0 else (NC, K, NPAD, HH)
    return pl.kernel(
        body,
        out_type=jax.ShapeDtypeStruct(oshape, jnp.float32),
        mesh=_mesh,
        scratch_types=scratch,
    )


_sc_hops_l0 = _make_hops(l0=True)
_sc_hops = _make_hops(l0=False)


_PBLK = NPAD // 8


def _tc_prep_body(deg2_ref, xp_ref, d2w_ref, sdeg_ref, dinv_ref, u0_ref):
    deg = deg2_ref[0, :, 0:1] + deg2_ref[1, :, 0:1]
    pos = deg > 0
    safe = jnp.maximum(deg, 1e-12)
    dinv = jnp.where(pos, lax.rsqrt(safe), 0.0)
    d2 = jnp.where(pos, 1.0 / safe, 0.0)
    d2w_ref[...] = jnp.broadcast_to(d2, (_PBLK, 16))
    sdeg_ref[...] = jnp.sqrt(deg)
    dinv_ref[...] = dinv
    u0_ref[...] = xp_ref[...] * dinv


_tc_prep = pl.pallas_call(
    _tc_prep_body,
    grid=(8,),
    in_specs=[
        pl.BlockSpec((NC, _PBLK, HH), lambda i: (0, i, 0)),
        pl.BlockSpec((_PBLK, HH), lambda i: (i, 0)),
    ],
    out_specs=[
        pl.BlockSpec((_PBLK, 16), lambda i: (i, 0)),
        pl.BlockSpec((_PBLK, 1), lambda i: (i, 0)),
        pl.BlockSpec((_PBLK, 1), lambda i: (i, 0)),
        pl.BlockSpec((_PBLK, HH), lambda i: (i, 0)),
    ],
    out_shape=[
        jax.ShapeDtypeStruct((NPAD, 16), jnp.float32),
        jax.ShapeDtypeStruct((NPAD, 1), jnp.float32),
        jax.ShapeDtypeStruct((NPAD, 1), jnp.float32),
        jax.ShapeDtypeStruct((NPAD, HH), jnp.float32),
    ],
)


def _tc_l0_body(xp_ref, u0_ref, sdeg_ref, dinv_ref, w_ref, b_ref,
                h_ref, u_ref):
    xs = xp_ref[:, :IN]
    acc = jnp.dot(xs, w_ref[0], preferred_element_type=jnp.float32)
    sd = sdeg_ref[...]
    for k in range(1, K + 1):
        uk = u0_ref[k - 1][:, :IN] * sd
        acc = acc + jnp.dot(uk, w_ref[k], preferred_element_type=jnp.float32)
    h = jnp.maximum(acc + b_ref[...], 0.0)
    h_ref[...] = h
    dv = dinv_ref[...]
    u_ref[0] = h[:, :HH] * dv
    u_ref[1] = h[:, HH:] * dv


_tc_l0 = pl.pallas_call(
    _tc_l0_body,
    grid=(8,),
    in_specs=[
        pl.BlockSpec((_PBLK, HH), lambda i: (i, 0)),
        pl.BlockSpec((K, _PBLK, HH), lambda i: (0, i, 0)),
        pl.BlockSpec((_PBLK, 1), lambda i: (i, 0)),
        pl.BlockSpec((_PBLK, 1), lambda i: (i, 0)),
        pl.BlockSpec((K + 1, IN, H), lambda i: (0, 0, 0)),
        pl.BlockSpec((1, H), lambda i: (0, 0)),
    ],
    out_specs=[
        pl.BlockSpec((_PBLK, H), lambda i: (i, 0)),
        pl.BlockSpec((NC, _PBLK, HH), lambda i: (0, i, 0)),
    ],
    out_shape=[
        jax.ShapeDtypeStruct((NPAD, H), jnp.float32),
        jax.ShapeDtypeStruct((NC, NPAD, HH), jnp.float32),
    ],
)


def _tc_mid_body(h_ref, u_ref, sdeg_ref, dinv_ref, w_ref, b_ref,
                 h_out, u_out):
    acc = jnp.dot(h_ref[...], w_ref[0], preferred_element_type=jnp.float32)
    sd = sdeg_ref[...]
    for k in range(1, K + 1):
        ua = u_ref[0, k - 1] * sd
        ub = u_ref[1, k - 1] * sd
        acc = acc + jnp.dot(ua, w_ref[k, :HH, :],
                            preferred_element_type=jnp.float32)
        acc = acc + jnp.dot(ub, w_ref[k, HH:, :],
                            preferred_element_type=jnp.float32)
    h = jnp.maximum(acc + b_ref[...], 0.0)
    h_out[...] = h
    dv = dinv_ref[...]
    u_out[0] = h[:, :HH] * dv
    u_out[1] = h[:, HH:] * dv


def _tc_fin_body(h_ref, u_ref, sdeg_ref, wf_ref, bf_ref, w_ref, b_ref,
                 y_ref):
    acc = jnp.dot(h_ref[...], w_ref[0], preferred_element_type=jnp.float32)
    sd = sdeg_ref[...]
    for k in range(1, K + 1):
        ua = u_ref[0, k - 1] * sd
        ub = u_ref[1, k - 1] * sd
        acc = acc + jnp.dot(ua, w_ref[k, :HH, :],
                            preferred_element_type=jnp.float32)
        acc = acc + jnp.dot(ub, w_ref[k, HH:, :],
                            preferred_element_type=jnp.float32)
    h = jnp.maximum(acc + b_ref[...], 0.0)
    y = jnp.dot(h, wf_ref[...], preferred_element_type=jnp.float32)
    y_ref[...] = jax.nn.sigmoid(y + bf_ref[...])


_mid_specs = [
    pl.BlockSpec((_PBLK, H), lambda i: (i, 0)),
    pl.BlockSpec((NC, K, _PBLK, HH), lambda i: (0, 0, i, 0)),
    pl.BlockSpec((_PBLK, 1), lambda i: (i, 0)),
]
_w_specs = [
    pl.BlockSpec((K + 1, H, H), lambda i: (0, 0, 0)),
    pl.BlockSpec((1, H), lambda i: (0, 0)),
]

_tc_mid = pl.pallas_call(
    _tc_mid_body,
    grid=(8,),
    in_specs=_mid_specs + [pl.BlockSpec((_PBLK, 1), lambda i: (i, 0))]
    + _w_specs,
    out_specs=[
        pl.BlockSpec((_PBLK, H), lambda i: (i, 0)),
        pl.BlockSpec((NC, _PBLK, HH), lambda i: (0, i, 0)),
    ],
    out_shape=[
        jax.ShapeDtypeStruct((NPAD, H), jnp.float32),
        jax.ShapeDtypeStruct((NC, NPAD, HH), jnp.float32),
    ],
)

_tc_fin = pl.pallas_call(
    _tc_fin_body,
    grid=(8,),
    in_specs=_mid_specs + [
        pl.BlockSpec((H, 1), lambda i: (0, 0)),
        pl.BlockSpec((1, 1), lambda i: (0, 0)),
    ] + _w_specs,
    out_specs=[pl.BlockSpec((_PBLK, 1), lambda i: (i, 0))],
    out_shape=[jax.ShapeDtypeStruct((NPAD, 1), jnp.float32)],
)


def kernel(x, edge_index, edge_attr, batch, W0, b0, W1, b1, W2, b2, Wf, bf):
    row = edge_index[0].reshape(NS, TPT)
    col = edge_index[1].reshape(NS, TPT)
    padi = jnp.zeros((NS, TPT_PAD - TPT), jnp.int32)
    rowp = jnp.concatenate([row, padi], axis=1).reshape(
        NS, NGRP, GRP, CHUNK)
    colp = jnp.concatenate([col, padi + N], axis=1).reshape(
        NS, NGRP, GRP, CHUNK)
    cold = edge_index[1].reshape(NC, NS, EPT)
    padd = jnp.full((NC, NS, DCHUNKS * DCH - EPT), N, jnp.int32)
    colpd = jnp.concatenate([cold, padd], axis=2).reshape(
        NC, NS, DCHUNKS, DCH)
    xp = jnp.zeros((NPAD, HH), x.dtype).at[:N, :IN].set(x)
    zeros128 = jnp.zeros((NPAD, HH), jnp.float32)
    ones128 = jnp.ones((DCH, HH), jnp.float32)

    deg2 = _sc_deg(colpd, zeros128, ones128)
    d2w, sdeg, dinv, u0p = _tc_prep(deg2, xp)
    d2flat = d2w.reshape(-1)

    u0k = _sc_hops_l0(rowp, colp, u0p, d2flat)
    h1, u1 = _tc_l0(xp, u0k, sdeg, dinv, W0, b0.reshape(1, H))

    u1k = _sc_hops(rowp, colp, u1, d2flat)
    h2, u2 = _tc_mid(h1, u1k, sdeg, dinv, W1, b1.reshape(1, H))

    u2k = _sc_hops(rowp, colp, u2, d2flat)
    (y,) = _tc_fin(h2, u2k, sdeg, Wf, bf.reshape(1, 1), W2, b2.reshape(1, H))

    return y[:N]
